# baseline (speedup 1.0000x reference)
import jax
import jax.numpy as jnp
from jax.experimental import pallas as pl

N_GROUPS = 9
N_PER_GROUP = 131072
C_IN = 64
C_OUT = 64
BLK = 8192
B8 = BLK // 8
NB = N_PER_GROUP // BLK


def _affine_kernel(x_ref, w_ref, b_ref, o_ref):
    x = x_ref[0].reshape(BLK, C_IN)
    w = w_ref[0]
    b = b_ref[0, 0]
    yt = jax.lax.dot_general(
        w, x, (((1,), (1,)), ((), ())), preferred_element_type=jnp.float32
    )
    o_ref[...] = yt + b[:, None]


def kernel(inputs, weights, bias):
    x4 = inputs.reshape(N_GROUPS, N_PER_GROUP // 8, 8, C_IN)
    bias3 = bias.reshape(N_GROUPS, 1, C_OUT)
    out_t = pl.pallas_call(
        _affine_kernel,
        grid=(N_GROUPS, NB),
        in_specs=[
            pl.BlockSpec((1, B8, 8, C_IN), lambda g, n: (g, n, 0, 0)),
            pl.BlockSpec((1, C_OUT, C_IN), lambda g, n: (g, 0, 0)),
            pl.BlockSpec((1, 1, C_OUT), lambda g, n: (g, 0, 0)),
        ],
        out_specs=pl.BlockSpec((C_OUT, BLK), lambda g, n: (0, g * NB + n)),
        out_shape=jax.ShapeDtypeStruct((C_OUT, N_GROUPS * N_PER_GROUP), jnp.float32),
    )(x4, weights, bias3)
    return out_t.T
